# baseline (speedup 1.0000x reference)
import functools

import jax
import jax.numpy as jnp
from jax import lax
from jax.experimental import pallas as pl
from jax.experimental.pallas import tpu as pltpu
from jax.experimental.pallas import tpu_sc as plsc

D = 64
SCALE = 8.0
NC, NS, L = 2, 16, 16
NW = NC * NS
IW = 128
C = 512
G = C // IW


def _emb_body(nchunks, idx_hbm, table_hbm, out_hbm, idx_v, rows_v, sem):
    wid = lax.axis_index("s") * NC + lax.axis_index("c")
    rows_per_w = nchunks * C
    base = wid * rows_per_w
    irow0 = pl.multiple_of(base // IW, 8)
    pltpu.sync_copy(idx_hbm.at[pl.ds(irow0, rows_per_w // IW)], idx_v)

    def chunk_body(ci, carry):
        row0 = pl.multiple_of(base + ci * C, 8)
        copies = [
            pltpu.async_copy(
                table_hbm.at[idx_v.at[ci * G + j]],
                rows_v.at[pl.ds(j * IW, IW)],
                sem,
            )
            for j in range(G)
        ]
        for cp in copies:
            cp.wait()

        def scale_row(r, c2):
            for k in range(D // L):
                sl = pl.ds(k * L, L)
                rows_v[r, sl] = rows_v[r, sl] * SCALE
            return c2

        lax.fori_loop(0, C, scale_row, 0, unroll=2)
        pltpu.sync_copy(rows_v, out_hbm.at[pl.ds(row0, C)])
        return carry

    lax.fori_loop(0, nchunks, chunk_body, 0)


def kernel(tokens, table):
    B = tokens.size
    assert B % (NW * C) == 0
    nchunks = B // (NW * C)
    idx = tokens.reshape(B // IW, IW).astype(jnp.int32)
    mesh = plsc.VectorSubcoreMesh(core_axis_name="c", subcore_axis_name="s")
    out = pl.kernel(
        functools.partial(_emb_body, nchunks),
        out_type=jax.ShapeDtypeStruct((B, D), jnp.float32),
        mesh=mesh,
        compiler_params=pltpu.CompilerParams(use_tc_tiling_on_sc=False),
        scratch_types=[
            pltpu.VMEM((nchunks * G, IW), jnp.int32),
            pltpu.VMEM((C, D), jnp.float32),
            pltpu.SemaphoreType.DMA,
        ],
    )(idx, table)
    return out.reshape(*tokens.shape, D)


# speedup vs baseline: 1.0663x; 1.0663x over previous
import functools

import jax
import jax.numpy as jnp
from jax import lax
from jax.experimental import pallas as pl
from jax.experimental.pallas import tpu as pltpu
from jax.experimental.pallas import tpu_sc as plsc

D = 64
SCALE = 8.0
NC, NS, L = 2, 16, 16
NW = NC * NS
IW = 128
C = 256
G = C // IW
NBUF = 4


def _emb_body(nchunks, idx_hbm, table_hbm, out_hbm, idx_v, bufs, gsems):
    wid = lax.axis_index("s") * NC + lax.axis_index("c")
    rows_per_w = nchunks * C
    base = wid * rows_per_w
    irow0 = pl.multiple_of(base // IW, 8)
    pltpu.sync_copy(idx_hbm.at[pl.ds(irow0, rows_per_w // IW)], idx_v)

    def fire(ci, b):
        for j in range(G):
            pltpu.async_copy(
                table_hbm.at[idx_v.at[ci * G + j]],
                bufs[b].at[pl.ds(j * IW, IW)],
                gsems[b],
            )

    for b in range(NBUF):
        fire(b, b)

    def round_body(g, carry):
        for b in range(NBUF):
            ci = g * NBUF + b
            row0 = pl.multiple_of(base + ci * C, 8)
            pltpu.make_async_copy(
                table_hbm.at[pl.ds(0, C)], bufs[b], gsems[b]
            ).wait()

            @plsc.parallel_loop(0, C, unroll=8)
            def scale_row(r):
                for k in range(D // L):
                    sl = pl.ds(k * L, L)
                    bufs[b][r, sl] = bufs[b][r, sl] * SCALE

            pltpu.sync_copy(bufs[b], out_hbm.at[pl.ds(row0, C)])

            nci = ci + NBUF

            @pl.when(nci < nchunks)
            def _():
                fire(nci, b)

        return carry

    lax.fori_loop(0, nchunks // NBUF, round_body, 0)


def kernel(tokens, table):
    B = tokens.size
    assert B % (NW * C * NBUF) == 0
    nchunks = B // (NW * C)
    idx = tokens.reshape(B // IW, IW).astype(jnp.int32)
    mesh = plsc.VectorSubcoreMesh(core_axis_name="c", subcore_axis_name="s")
    out = pl.kernel(
        functools.partial(_emb_body, nchunks),
        out_type=jax.ShapeDtypeStruct((B, D), jnp.float32),
        mesh=mesh,
        compiler_params=pltpu.CompilerParams(use_tc_tiling_on_sc=False),
        scratch_types=[
            pltpu.VMEM((nchunks * G, IW), jnp.int32),
            [pltpu.VMEM((C, D), jnp.float32) for _ in range(NBUF)],
            [pltpu.SemaphoreType.DMA for _ in range(NBUF)],
        ],
    )(idx, table)
    return out.reshape(*tokens.shape, D)


# speedup vs baseline: 1.1659x; 1.0935x over previous
import jax
import jax.numpy as jnp
from jax import lax
from jax.experimental import pallas as pl
from jax.experimental.pallas import tpu as pltpu
from jax.experimental.pallas import tpu_sc as plsc

D = 64
SCALE = 8.0
NC, NS, L = 2, 16, 16
NW = NC * NS
BW = 128
NBUF = 4


def _emb_body(S, idx_hbm, table_hbm, out_hbm, idx_v, rowbufs, tbufs, gsems, wsems):
    wid = lax.axis_index("s") * NC + lax.axis_index("c")
    b0 = wid * BW
    pltpu.sync_copy(idx_hbm.at[:, pl.ds(b0, BW)], idx_v)

    iota = lax.iota(jnp.int32, L)
    dvecs = [jnp.full((L,), 16 * k, jnp.int32) + iota for k in range(4)]

    def fire(s, slot):
        pltpu.async_copy(table_hbm.at[idx_v.at[s]], rowbufs[slot], gsems[slot])

    for slot in range(NBUF):
        fire(slot, slot)

    def round_body(g, carry):
        for u in range(NBUF):
            s = g * NBUF + u
            slot = u
            tsl = u % 2
            pltpu.make_async_copy(
                table_hbm.at[pl.ds(0, BW)], rowbufs[slot], gsems[slot]
            ).wait()

            if u >= 2:
                for i in range(D // 8):
                    pltpu.make_async_copy(
                        tbufs[tsl].at[pl.ds(8 * i, 8), pl.ds(0, BW)],
                        out_hbm.at[s - 2, i, wid],
                        wsems[tsl],
                    ).wait()
            else:

                @pl.when(g > 0)
                def _():
                    for i in range(D // 8):
                        pltpu.make_async_copy(
                            tbufs[tsl].at[pl.ds(8 * i, 8), pl.ds(0, BW)],
                            out_hbm.at[s - 2, i, wid],
                            wsems[tsl],
                        ).wait()

            def transpose_scale(b, c2):
                colv = jnp.full((L,), b, jnp.int32)
                for k in range(4):
                    v = rowbufs[slot][b, pl.ds(16 * k, L)] * SCALE
                    plsc.store_scatter(tbufs[tsl], [dvecs[k], colv], v)
                return c2

            lax.fori_loop(0, BW, transpose_scale, 0)

            for i in range(D // 8):
                pltpu.async_copy(
                    tbufs[tsl].at[pl.ds(8 * i, 8), pl.ds(0, BW)],
                    out_hbm.at[s, i, wid],
                    wsems[tsl],
                )

            @pl.when(s + NBUF < S)
            def _():
                fire(s + NBUF, slot)

        return carry

    lax.fori_loop(0, S // NBUF, round_body, 0)

    for tsl in range(2):
        for i in range(D // 8):
            pltpu.make_async_copy(
                tbufs[tsl].at[pl.ds(8 * i, 8), pl.ds(0, BW)],
                out_hbm.at[S - 2 + tsl, i, wid],
                wsems[tsl],
            ).wait()


def kernel(tokens, table):
    B, S = tokens.shape
    assert B == NW * BW and S % NBUF == 0
    idx = tokens.T.astype(jnp.int32)
    mesh = plsc.VectorSubcoreMesh(core_axis_name="c", subcore_axis_name="s")
    out5 = pl.kernel(
        lambda *refs: _emb_body(S, *refs),
        out_type=jax.ShapeDtypeStruct((S, D // 8, B // 128, 8, 128), jnp.float32),
        mesh=mesh,
        compiler_params=pltpu.CompilerParams(
            use_tc_tiling_on_sc=False, needs_layout_passes=False
        ),
        scratch_types=[
            pltpu.VMEM((S, BW), jnp.int32),
            [pltpu.VMEM((BW, D), jnp.float32) for _ in range(NBUF)],
            [pltpu.VMEM((D, 129), jnp.float32) for _ in range(2)],
            [pltpu.SemaphoreType.DMA for _ in range(NBUF)],
            [pltpu.SemaphoreType.DMA for _ in range(2)],
        ],
    )(idx, table)
    return out5.transpose(2, 4, 0, 1, 3).reshape(B, S, D)


# speedup vs baseline: 1.7575x; 1.5074x over previous
import jax
import jax.numpy as jnp
from jax import lax
from jax.experimental import pallas as pl
from jax.experimental.pallas import tpu as pltpu
from jax.experimental.pallas import tpu_sc as plsc

D = 64
SCALE = 8.0
NC, NS, L = 2, 16, 16
NW = NC * NS
BW = 128
NBUF = 4


def _emb_body(S, idx_hbm, table_hbm, out_hbm, idx_v, rowbufs, tbufs, gsems, wsems):
    wid = lax.axis_index("s") * NC + lax.axis_index("c")
    b0 = wid * BW
    pltpu.sync_copy(idx_hbm.at[:, pl.ds(b0, BW)], idx_v)

    iota = lax.iota(jnp.int32, L)
    dvecs = [jnp.full((L,), 16 * k, jnp.int32) + iota for k in range(4)]

    def fire(s, slot):
        pltpu.async_copy(table_hbm.at[idx_v.at[s]], rowbufs[slot], gsems[slot])

    for slot in range(NBUF):
        fire(slot, slot)

    def round_body(g, carry):
        for u in range(NBUF):
            s = g * NBUF + u
            slot = u
            tsl = u % 2
            pltpu.make_async_copy(
                table_hbm.at[pl.ds(0, BW)], rowbufs[slot], gsems[slot]
            ).wait()

            if u >= 2:
                for i in range(D // 8):
                    pltpu.make_async_copy(
                        tbufs[tsl].at[pl.ds(8 * i, 8), pl.ds(0, BW)],
                        out_hbm.at[s - 2, i, wid],
                        wsems[tsl],
                    ).wait()
            else:

                @pl.when(g > 0)
                def _():
                    for i in range(D // 8):
                        pltpu.make_async_copy(
                            tbufs[tsl].at[pl.ds(8 * i, 8), pl.ds(0, BW)],
                            out_hbm.at[s - 2, i, wid],
                            wsems[tsl],
                        ).wait()

            @plsc.parallel_loop(0, BW, unroll=8)
            def transpose_scale(b):
                colv = jnp.full((L,), b, jnp.int32)
                for k in range(4):
                    v = rowbufs[slot][b, pl.ds(16 * k, L)] * SCALE
                    plsc.store_scatter(tbufs[tsl], [dvecs[k], colv], v)

            for i in range(D // 8):
                pltpu.async_copy(
                    tbufs[tsl].at[pl.ds(8 * i, 8), pl.ds(0, BW)],
                    out_hbm.at[s, i, wid],
                    wsems[tsl],
                )

            @pl.when(s + NBUF < S)
            def _():
                fire(s + NBUF, slot)

        return carry

    lax.fori_loop(0, S // NBUF, round_body, 0)

    for tsl in range(2):
        for i in range(D // 8):
            pltpu.make_async_copy(
                tbufs[tsl].at[pl.ds(8 * i, 8), pl.ds(0, BW)],
                out_hbm.at[S - 2 + tsl, i, wid],
                wsems[tsl],
            ).wait()


def kernel(tokens, table):
    B, S = tokens.shape
    assert B == NW * BW and S % NBUF == 0
    idx = tokens.T.astype(jnp.int32)
    mesh = plsc.VectorSubcoreMesh(core_axis_name="c", subcore_axis_name="s")
    out5 = pl.kernel(
        lambda *refs: _emb_body(S, *refs),
        out_type=jax.ShapeDtypeStruct((S, D // 8, B // 128, 8, 128), jnp.float32),
        mesh=mesh,
        compiler_params=pltpu.CompilerParams(
            use_tc_tiling_on_sc=False, needs_layout_passes=False
        ),
        scratch_types=[
            pltpu.VMEM((S, BW), jnp.int32),
            [pltpu.VMEM((BW, D), jnp.float32) for _ in range(NBUF)],
            [pltpu.VMEM((D, 129), jnp.float32) for _ in range(2)],
            [pltpu.SemaphoreType.DMA for _ in range(NBUF)],
            [pltpu.SemaphoreType.DMA for _ in range(2)],
        ],
    )(idx, table)
    return out5.transpose(2, 4, 0, 1, 3).reshape(B, S, D)
